# Initial kernel scaffold; baseline (speedup 1.0000x reference)
import functools

import jax
import jax.numpy as jnp
import numpy as np
from jax import lax
from jax.experimental import pallas as pl
from jax.experimental.pallas import tpu as pltpu
from jax.experimental.pallas import tpu_sc as plsc

GX = GY = GZ = 128
NUM_RAYS = 4096
NUM_SAMPLES = 64
VOXEL_DIM = 28

Y_0_0 = 0.5 * np.sqrt(1.0 / np.pi)
HALF_SQRT_3_BY_PI = 0.5 * np.sqrt(3.0 / np.pi)
QUARTER_SQRT_5_BY_PI = 0.25 * np.sqrt(5.0 / np.pi)
HALF_SQRT_15_BY_PI = 0.5 * np.sqrt(15.0 / np.pi)
QUARTER_SQRT_15_BY_PI = 0.25 * np.sqrt(15.0 / np.pi)

NW = 32
RPT = NUM_RAYS // NW
NG = NUM_SAMPLES // 16


def _basis_tc(viewing_angles):
    def body(va_ref, out_ref):
        th = va_ref[:, 0:1]
        ph = va_ref[:, 1:2]
        st, ct = jnp.sin(th), jnp.cos(th)
        sp, cp = jnp.sin(ph), jnp.cos(ph)
        cols = [
            jnp.full_like(th, Y_0_0),
            HALF_SQRT_3_BY_PI * st * sp,
            HALF_SQRT_3_BY_PI * ct,
            HALF_SQRT_3_BY_PI * st * cp,
            HALF_SQRT_15_BY_PI * st * cp * st * sp,
            HALF_SQRT_15_BY_PI * st * sp * ct,
            QUARTER_SQRT_5_BY_PI * (3.0 * ct * ct - 1.0),
            HALF_SQRT_15_BY_PI * st * cp * ct,
            QUARTER_SQRT_15_BY_PI * ((st * cp) ** 2 - (st * sp) ** 2),
        ]
        li = lax.broadcasted_iota(jnp.int32, (NUM_RAYS, 16), 1)
        acc = jnp.zeros((NUM_RAYS, 16), jnp.float32)
        for k, c in enumerate(cols):
            acc += jnp.where(li == k, c, 0.0)
        out_ref[:, :] = acc

    return pl.pallas_call(
        body,
        out_shape=jax.ShapeDtypeStruct((NUM_RAYS, 16), jnp.float32),
    )(viewing_angles)


def _sc_render(flat_grid, positions, distances, basis):
    mesh = plsc.VectorSubcoreMesh(
        core_axis_name="c", subcore_axis_name="s", num_cores=2, num_subcores=16
    )

    @functools.partial(
        pl.kernel,
        out_type=jax.ShapeDtypeStruct((NUM_RAYS, 3), jnp.float32),
        mesh=mesh,
        scratch_types=[
            pltpu.VMEM((RPT, NUM_SAMPLES, 3), jnp.float32),
            pltpu.VMEM((RPT, NUM_SAMPLES), jnp.float32),
            pltpu.VMEM((RPT, 16), jnp.float32),
            pltpu.VMEM((2, 512), jnp.float32),
            pltpu.VMEM((2, 4, 128), jnp.int32),
            pltpu.VMEM((2, 512, VOXEL_DIM), jnp.float32),
            pltpu.VMEM((RPT, 3), jnp.float32),
            pltpu.SemaphoreType.DMA,
            pltpu.SemaphoreType.DMA,
        ],
    )
    def k(grid, pos, dist, bas, out, pos_v, dist_v, basis_v, wbuf, idxb,
          rows_v, out_v, sem0, sem1):
        wid = lax.axis_index("s") * 2 + lax.axis_index("c")
        ray0 = wid * RPT
        pltpu.sync_copy(pos.at[pl.ds(ray0, RPT)], pos_v)
        pltpu.sync_copy(dist.at[pl.ds(ray0, RPT)], dist_v)
        pltpu.sync_copy(bas.at[pl.ds(ray0, RPT)], basis_v)

        lane = lax.iota(jnp.int32, 16)
        sems = (sem0, sem1)

        def build(ray, b):
            rayv = jnp.full((16,), ray, jnp.int32)
            for g in range(NG):
                sv = lane + 16 * g
                x = plsc.load_gather(pos_v, [rayv, sv, jnp.zeros((16,), jnp.int32)])
                y = plsc.load_gather(pos_v, [rayv, sv, jnp.full((16,), 1, jnp.int32)])
                z = plsc.load_gather(pos_v, [rayv, sv, jnp.full((16,), 2, jnp.int32)])
                xi = x.astype(jnp.int32)
                yi = y.astype(jnp.int32)
                zi = z.astype(jnp.int32)
                xd = x - xi.astype(jnp.float32)
                yd = y - yi.astype(jnp.float32)
                zd = z - zi.astype(jnp.float32)
                vid = xi * (GY * GZ) + yi * GZ + zi
                wx = (1.0 - xd, xd)
                wy = (1.0 - yd, yd)
                wz = (1.0 - zd, zd)
                for r in range(8):
                    dx, dy, dz = (r >> 2) & 1, (r >> 1) & 1, r & 1
                    off = r * 64 + 16 * g
                    idxb[b, off >> 7, pl.ds(off & 127, 16)] = (
                        vid + (dx * (GY * GZ) + dy * GZ + dz)
                    )
                    wbuf[b, pl.ds(off, 16)] = wx[dx] * wy[dy] * wz[dz]

        def fire(b):
            for i in range(4):
                pltpu.async_copy(
                    grid.at[idxb.at[b, i]],
                    rows_v.at[b, pl.ds(i * 128, 128)],
                    sems[b],
                )

        def drain(b):
            for i in range(4):
                pltpu.make_async_copy(
                    grid.at[idxb.at[b, i]],
                    rows_v.at[b, pl.ds(i * 128, 128)],
                    sems[b],
                ).wait()

        def blend(ray, b):
            rayv = jnp.full((16,), ray, jnp.int32)
            bk = [
                plsc.load_gather(basis_v, [rayv, jnp.full((16,), kk, jnp.int32)])
                for kk in range(9)
            ]
            rowsb = rows_v.at[b]
            racc = jnp.float32(0.0)
            gacc = jnp.float32(0.0)
            bacc = jnp.float32(0.0)
            carry = jnp.float32(0.0)
            for g in range(NG):
                s0 = 16 * g
                riv = [lane + (r * 64 + s0) for r in range(8)]
                wv = [wbuf[b, pl.ds(r * 64 + s0, 16)] for r in range(8)]

                def chan(j):
                    jv = jnp.full((16,), j, jnp.int32)
                    acc = wv[0] * plsc.load_gather(rowsb, [riv[0], jv])
                    for r in range(1, 8):
                        acc += wv[r] * plsc.load_gather(rowsb, [riv[r], jv])
                    return acc

                sig = chan(0)
                cols = []
                for c in range(3):
                    col = bk[0] * chan(1 + 9 * c)
                    for kk in range(1, 9):
                        col += bk[kk] * chan(1 + 9 * c + kk)
                    cols.append(col)
                d_g = dist_v[ray, pl.ds(s0, 16)]
                att = jnp.exp(-sig * d_g)
                csum = plsc.cumsum(att) + carry
                w = csum * (1.0 - att)
                wm = jnp.where(sig != 0.0, w, 0.0)
                racc += jnp.sum(wm * cols[0])
                gacc += jnp.sum(wm * cols[1])
                bacc += jnp.sum(wm * cols[2])
                carry += jnp.sum(att)
            rgbv = jnp.where(lane == 0, racc, jnp.where(lane == 1, gacc, bacc))
            plsc.store_scatter(out_v, [rayv, lane], rgbv, mask=lane < 3)

        build(jnp.int32(0), 0)
        fire(0)

        def body(i, c):
            r0 = 2 * i
            build(r0 + 1, 1)
            fire(1)
            drain(0)
            blend(r0, 0)

            @pl.when(i < (RPT // 2 - 1))
            def _():
                build(r0 + 2, 0)
                fire(0)

            drain(1)
            blend(r0 + 1, 1)
            return c

        lax.fori_loop(0, RPT // 2, body, jnp.int32(0))
        pltpu.sync_copy(out_v, out.at[pl.ds(ray0, RPT)])

    return k(flat_grid, positions, distances, basis)


def kernel(positions, distances, viewing_angles, voxel_grid):
    basis = _basis_tc(viewing_angles)
    flat = voxel_grid.reshape(-1, VOXEL_DIM)
    return _sc_render(flat, positions, distances, basis)


# speedup vs baseline: 2.1873x; 2.1873x over previous
import functools

import jax
import jax.numpy as jnp
import numpy as np
from jax import lax
from jax.experimental import pallas as pl
from jax.experimental.pallas import tpu as pltpu
from jax.experimental.pallas import tpu_sc as plsc

GX = GY = GZ = 128
NUM_RAYS = 4096
NUM_SAMPLES = 64
VOXEL_DIM = 28

Y_0_0 = 0.5 * np.sqrt(1.0 / np.pi)
HALF_SQRT_3_BY_PI = 0.5 * np.sqrt(3.0 / np.pi)
QUARTER_SQRT_5_BY_PI = 0.25 * np.sqrt(5.0 / np.pi)
HALF_SQRT_15_BY_PI = 0.5 * np.sqrt(15.0 / np.pi)
QUARTER_SQRT_15_BY_PI = 0.25 * np.sqrt(15.0 / np.pi)

NW = 32
RPT = NUM_RAYS // NW
NG = NUM_SAMPLES // 16
NROWS16 = GX * GY * GZ * VOXEL_DIM // 16
IDX_PER_RAY = 4 * 5 * NUM_SAMPLES


def _basis_tc(viewing_angles):
    def body(va_ref, out_ref):
        th = va_ref[:, 0:1]
        ph = va_ref[:, 1:2]
        st, ct = jnp.sin(th), jnp.cos(th)
        sp, cp = jnp.sin(ph), jnp.cos(ph)
        cols = [
            jnp.full_like(th, Y_0_0),
            HALF_SQRT_3_BY_PI * st * sp,
            HALF_SQRT_3_BY_PI * ct,
            HALF_SQRT_3_BY_PI * st * cp,
            HALF_SQRT_15_BY_PI * st * cp * st * sp,
            HALF_SQRT_15_BY_PI * st * sp * ct,
            QUARTER_SQRT_5_BY_PI * (3.0 * ct * ct - 1.0),
            HALF_SQRT_15_BY_PI * st * cp * ct,
            QUARTER_SQRT_15_BY_PI * ((st * cp) ** 2 - (st * sp) ** 2),
        ]
        li = lax.broadcasted_iota(jnp.int32, (NUM_RAYS, 16), 1)
        acc = jnp.zeros((NUM_RAYS, 16), jnp.float32)
        for k, c in enumerate(cols):
            acc += jnp.where(li == k, c, 0.0)
        out_ref[:, :] = acc

    return pl.pallas_call(
        body,
        out_shape=jax.ShapeDtypeStruct((NUM_RAYS, 16), jnp.float32),
    )(viewing_angles)


def _sc_render(tab16, positions, distances, basis):
    mesh = plsc.VectorSubcoreMesh(
        core_axis_name="c", subcore_axis_name="s", num_cores=2, num_subcores=16
    )

    @functools.partial(
        pl.kernel,
        out_type=jax.ShapeDtypeStruct((NUM_RAYS, 3), jnp.float32),
        mesh=mesh,
        compiler_params=pltpu.CompilerParams(
            use_tc_tiling_on_sc=False, needs_layout_passes=False
        ),
        scratch_types=[
            pltpu.VMEM((RPT, NUM_SAMPLES * 3), jnp.float32),
            pltpu.VMEM((RPT, NUM_SAMPLES), jnp.float32),
            pltpu.VMEM((RPT, 16), jnp.float32),
            pltpu.VMEM((2, 512), jnp.float32),
            pltpu.VMEM((2, 256), jnp.int32),
            pltpu.VMEM((2, IDX_PER_RAY), jnp.int32),
            pltpu.VMEM((2, IDX_PER_RAY, 16), jnp.float32),
            pltpu.VMEM((RPT, 3), jnp.float32),
            pltpu.SemaphoreType.DMA,
            pltpu.SemaphoreType.DMA,
        ],
    )
    def k(grid, pos, dist, bas, out, pos_v, dist_v, basis_v, wbuf, obuf,
          idxb, rows_v, out_v, sem0, sem1):
        wid = lax.axis_index("s") * 2 + lax.axis_index("c")
        ray0 = wid * RPT
        pltpu.sync_copy(pos.at[pl.ds(ray0, RPT)], pos_v)
        pltpu.sync_copy(dist.at[pl.ds(ray0, RPT)], dist_v)
        pltpu.sync_copy(bas.at[pl.ds(ray0, RPT)], basis_v)

        lane = lax.iota(jnp.int32, 16)
        sems = (sem0, sem1)

        def build(ray, b):
            rayv = jnp.full((16,), ray, jnp.int32)

            def grp(g, c):
                s0 = 16 * g
                sv = (lane + s0) * 3
                x = plsc.load_gather(pos_v, [rayv, sv])
                y = plsc.load_gather(pos_v, [rayv, sv + 1])
                z = plsc.load_gather(pos_v, [rayv, sv + 2])
                xi = x.astype(jnp.int32)
                yi = y.astype(jnp.int32)
                zi = z.astype(jnp.int32)
                xd = x - xi.astype(jnp.float32)
                yd = y - yi.astype(jnp.float32)
                zd = z - zi.astype(jnp.float32)
                vid = xi * (GY * GZ) + yi * GZ + zi
                wx = (1.0 - xd, xd)
                wy = (1.0 - yd, yd)
                wz = (1.0 - zd, zd)
                for p in range(4):
                    dx, dy = (p >> 1) & 1, p & 1
                    vp = vid + (dx * (GY * GZ) + dy * GZ)
                    w28 = vp * VOXEL_DIM
                    r16 = lax.shift_right_logical(w28, 4)
                    al = w28 & 15
                    slot = p * 64 + s0 + lane
                    slot5 = slot * 5
                    for q in range(5):
                        plsc.store_scatter(idxb.at[b], [slot5 + q], r16 + q)
                    obuf[b, pl.ds(p * 64 + s0, 16)] = slot * 80 + al
                    for dz in range(2):
                        woff = (p * 2 + dz) * 64 + s0
                        wbuf[b, pl.ds(woff, 16)] = wx[dx] * wy[dy] * wz[dz]
                return c

            lax.fori_loop(0, NG, grp, jnp.int32(0))

        def fire(b):
            for i in range(IDX_PER_RAY // 128):
                pltpu.async_copy(
                    grid.at[idxb.at[b, pl.ds(i * 128, 128)]],
                    rows_v.at[b, pl.ds(i * 128, 128)],
                    sems[b],
                )

        def drain(b):
            for i in range(IDX_PER_RAY // 128):
                pltpu.make_async_copy(
                    grid.at[idxb.at[b, pl.ds(i * 128, 128)]],
                    rows_v.at[b, pl.ds(i * 128, 128)],
                    sems[b],
                ).wait()

        def blend(ray, b):
            rayv = jnp.full((16,), ray, jnp.int32)
            bk = [
                plsc.load_gather(basis_v, [rayv, jnp.full((16,), kk, jnp.int32)])
                for kk in range(9)
            ]
            rowsb = rows_v.at[b]

            def grp(g, carry4):
                racc, gacc, bacc, csum_c = carry4
                s0 = 16 * g
                base = [obuf[b, pl.ds(p * 64 + s0, 16)] for p in range(4)]
                wv = [wbuf[b, pl.ds(r * 64 + s0, 16)] for r in range(8)]

                def chan(j):
                    acc = None
                    for p in range(4):
                        for dz in range(2):
                            t = base[p] + (VOXEL_DIM * dz + j)
                            row = lax.shift_right_logical(t, 4)
                            col = t & 15
                            v = plsc.load_gather(rowsb, [row, col])
                            term = wv[p * 2 + dz] * v
                            acc = term if acc is None else acc + term
                    return acc

                sig = chan(0)
                cols = []
                for c in range(3):
                    col = bk[0] * chan(1 + 9 * c)
                    for kk in range(1, 9):
                        col += bk[kk] * chan(1 + 9 * c + kk)
                    cols.append(col)
                d_g = dist_v[ray, pl.ds(s0, 16)]
                att = jnp.exp(-sig * d_g)
                csum = plsc.cumsum(att) + csum_c
                w = csum * (1.0 - att)
                wm = jnp.where(sig != 0.0, w, 0.0)
                return (
                    racc + jnp.sum(wm * cols[0]),
                    gacc + jnp.sum(wm * cols[1]),
                    bacc + jnp.sum(wm * cols[2]),
                    csum_c + jnp.sum(att),
                )

            z = jnp.float32(0.0)
            racc, gacc, bacc, _ = lax.fori_loop(0, NG, grp, (z, z, z, z))
            rgbv = jnp.where(lane == 0, racc, jnp.where(lane == 1, gacc, bacc))
            plsc.store_scatter(out_v, [rayv, lane], rgbv, mask=lane < 3)

        build(jnp.int32(0), 0)
        fire(0)

        def body(i, c):
            r0 = 2 * i
            build(r0 + 1, 1)
            fire(1)
            drain(0)
            blend(r0, 0)

            @pl.when(i < (RPT // 2 - 1))
            def _():
                build(r0 + 2, 0)
                fire(0)

            drain(1)
            blend(r0 + 1, 1)
            return c

        lax.fori_loop(0, RPT // 2, body, jnp.int32(0))
        pltpu.sync_copy(out_v, out.at[pl.ds(ray0, RPT)])

    return k(tab16, positions, distances, basis)


def kernel(positions, distances, viewing_angles, voxel_grid):
    basis = _basis_tc(viewing_angles)
    tab16 = voxel_grid.reshape(-1, 16)
    pos2d = positions.reshape(NUM_RAYS, NUM_SAMPLES * 3)
    return _sc_render(tab16, pos2d, distances, basis)
